# Initial kernel scaffold; baseline (speedup 1.0000x reference)
import jax
import jax.numpy as jnp
from jax.experimental import pallas as pl


def kernel(qkvp, pfa_values, pfa_indices, rpi, rpb_table, proj_w, proj_b, shift):
    raise NotImplementedError("write your pallas kernel here")


# speedup vs baseline: 69.0282x; 69.0282x over previous
import functools

import jax
import jax.numpy as jnp
from jax import lax
from jax.experimental import pallas as pl
from jax.experimental.pallas import tpu as pltpu
from jax.experimental.pallas import tpu_sc as plsc


def _scores_kernel(qkvp_ref, s_ref, *, c, H, dh, scale):
    blk = qkvp_ref[0]
    for h in range(H):
        q = blk[:, h * dh:(h + 1) * dh]
        k = blk[:, c + h * dh:c + (h + 1) * dh]
        s = lax.dot_general(q, k, (((1,), (1,)), ((), ())),
                            preferred_element_type=jnp.float32)
        s_ref[0, h] = s * scale


def _make_scores(b_, n, c, H, dh, scale):
    return pl.pallas_call(
        functools.partial(_scores_kernel, c=c, H=H, dh=dh, scale=scale),
        grid=(b_,),
        in_specs=[
            pl.BlockSpec((1, n, 4 * c), lambda b: (b, 0, 0)),
        ],
        out_specs=pl.BlockSpec((1, H, n, n), lambda b: (b, 0, 0, 0)),
        out_shape=jax.ShapeDtypeStruct((b_, H, n, n), jnp.float32),
    )


def _sc_attn_body(s_hbm, idx_hbm, pv_hbm, rpi_hbm, rpb_hbm, w_hbm,
                  s_v, rpi_v, rpb_v, idx_v, pv_v, w_v,
                  *, b_, n, H, topk, rows):
    cid = lax.axis_index("c")
    sid = lax.axis_index("s")
    wid = sid * 2 + cid
    i0 = wid * rows

    pltpu.sync_copy(rpi_hbm.at[pl.ds(i0, rows), :], rpi_v)
    pltpu.sync_copy(rpb_hbm, rpb_v)

    zeros16 = jnp.zeros((16,), jnp.float32)
    for r in range(rows):
        for j0 in range(0, n, 16):
            w_v[r, j0:j0 + 16] = zeros16

    ngroups = topk // 16

    def bh_body(bh, carry):
        b = bh // H
        h = bh % H
        pltpu.sync_copy(idx_hbm.at[b, h, pl.ds(i0, rows), :], idx_v)
        pltpu.sync_copy(pv_hbm.at[b, h, pl.ds(i0, rows), :], pv_v)
        pltpu.sync_copy(s_hbm.at[b, h, pl.ds(i0, rows), :], s_v)
        h16 = jnp.full((16,), h, jnp.int32)
        for r in range(rows):
            r16 = jnp.full((16,), r, jnp.int32)
            ws = []
            idxs = []
            tot = zeros16
            for g in range(ngroups):
                idx_g = idx_v[r, g * 16:(g + 1) * 16]
                s_g = plsc.load_gather(s_v, [r16, idx_g])
                ri_g = plsc.load_gather(rpi_v, [r16, idx_g])
                rb_g = plsc.load_gather(rpb_v, [ri_g, h16])
                e_g = jnp.exp(s_g + rb_g) * pv_v[r, g * 16:(g + 1) * 16]
                tot = tot + e_g
                ws.append(e_g)
                idxs.append(idx_g)
            den = jnp.full((16,), jnp.sum(tot) + 1e-20, jnp.float32)
            for g in range(ngroups):
                plsc.addupdate_scatter(w_v, [r16, idxs[g]], ws[g] / den)
        pltpu.sync_copy(w_v, w_hbm.at[b, h, pl.ds(i0, rows), :])
        for r in range(rows):
            r16 = jnp.full((16,), r, jnp.int32)
            for g in range(ngroups):
                plsc.store_scatter(w_v, [r16, idx_v[r, g * 16:(g + 1) * 16]],
                                   zeros16)
        return carry

    lax.fori_loop(0, b_ * H, bh_body, 0)


def _make_sc_attn(b_, n, H, topk, nrpb):
    rows = n // 32
    mesh = plsc.VectorSubcoreMesh(core_axis_name="c", subcore_axis_name="s",
                                  num_cores=2, num_subcores=16)
    return pl.kernel(
        functools.partial(_sc_attn_body, b_=b_, n=n, H=H, topk=topk,
                          rows=rows),
        out_type=jax.ShapeDtypeStruct((b_, H, n, n), jnp.float32),
        mesh=mesh,
        compiler_params=pltpu.CompilerParams(use_tc_tiling_on_sc=False,
                                             needs_layout_passes=False),
        scratch_types=[
            pltpu.VMEM((rows, n), jnp.float32),
            pltpu.VMEM((rows, n), jnp.int32),
            pltpu.VMEM((nrpb, H), jnp.float32),
            pltpu.VMEM((rows, topk), jnp.int32),
            pltpu.VMEM((rows, topk), jnp.float32),
            pltpu.VMEM((rows, n), jnp.float32),
        ],
    )


def _out_kernel(w_ref, qkvp_ref, pw_ref, pb_ref, o_ref, *, c, H, dh):
    blk = qkvp_ref[0]
    parts = []
    for h in range(H):
        v = blk[:, 2 * c + h * dh:2 * c + (h + 1) * dh]
        parts.append(lax.dot_general(
            w_ref[0, h], v,
            (((1,), (0,)), ((), ())), preferred_element_type=jnp.float32))
    x = jnp.concatenate(parts, axis=1) + blk[:, 3 * c:4 * c]
    o = lax.dot_general(x, pw_ref[...], (((1,), (1,)), ((), ())),
                        preferred_element_type=jnp.float32)
    o_ref[0] = o + pb_ref[...]


def _make_out(b_, n, c, H, dh):
    return pl.pallas_call(
        functools.partial(_out_kernel, c=c, H=H, dh=dh),
        grid=(b_,),
        in_specs=[
            pl.BlockSpec((1, H, n, n), lambda b: (b, 0, 0, 0)),
            pl.BlockSpec((1, n, 4 * c), lambda b: (b, 0, 0)),
            pl.BlockSpec((c, c), lambda b: (0, 0)),
            pl.BlockSpec((1, c), lambda b: (0, 0)),
        ],
        out_specs=pl.BlockSpec((1, n, c), lambda b: (b, 0, 0)),
        out_shape=jax.ShapeDtypeStruct((b_, n, c), jnp.float32),
    )


def kernel(qkvp, pfa_values, pfa_indices, rpi, rpb_table, proj_w, proj_b,
           shift):
    b_, n, c4 = qkvp.shape
    c = c4 // 4
    H = rpb_table.shape[1]
    dh = c // H
    topk = pfa_indices.shape[-1]
    nrpb = rpb_table.shape[0]
    scale = dh ** (-0.5)

    idx = lax.dynamic_index_in_dim(pfa_indices, shift, 0, keepdims=False)
    pv = lax.dynamic_index_in_dim(pfa_values, shift, 0, keepdims=False)

    s = _make_scores(b_, n, c, H, dh, scale)(qkvp)
    w = _make_sc_attn(b_, n, H, topk, nrpb)(
        s, idx, pv, rpi.astype(jnp.int32), rpb_table)
    out = _make_out(b_, n, c, H, dh)(
        w, qkvp, proj_w, proj_b.reshape(1, c))
    return out


# speedup vs baseline: 109.5322x; 1.5868x over previous
import functools

import jax
import jax.numpy as jnp
from jax import lax
from jax.experimental import pallas as pl
from jax.experimental.pallas import tpu as pltpu
from jax.experimental.pallas import tpu_sc as plsc


def _scores_kernel(qkvp_ref, s_ref, *, c, H, dh, scale):
    blk = qkvp_ref[0]
    for h in range(H):
        q = blk[:, h * dh:(h + 1) * dh]
        k = blk[:, c + h * dh:c + (h + 1) * dh]
        s = lax.dot_general(q, k, (((1,), (1,)), ((), ())),
                            preferred_element_type=jnp.float32)
        s_ref[0, h] = s * scale


def _make_scores(b_, n, c, H, dh, scale):
    return pl.pallas_call(
        functools.partial(_scores_kernel, c=c, H=H, dh=dh, scale=scale),
        grid=(b_,),
        in_specs=[
            pl.BlockSpec((1, n, 4 * c), lambda b: (b, 0, 0)),
        ],
        out_specs=pl.BlockSpec((1, H, n, n), lambda b: (b, 0, 0, 0)),
        out_shape=jax.ShapeDtypeStruct((b_, H, n, n), jnp.float32),
    )


def _sc_attn_body(s_hbm, idx_hbm, pv_hbm, rpi_hbm, rpb_hbm, shift_hbm, w_hbm,
                  s_v0, s_v1,
                  ix0a, ix0b, ix1a, ix1b, ix2a, ix2b, ix3a, ix3b,
                  pv0a, pv0b, pv1a, pv1b,
                  w_v0, w_v1, rpi_v, rpb_v, shift_v,
                  sem_s0, sem_s1, sem_i0, sem_i1, sem_p0, sem_p1,
                  sem_o0, sem_o1,
                  *, b_, n, H, topk, rows):
    cid = lax.axis_index("c")
    sid = lax.axis_index("s")
    wid = sid * 2 + cid
    i0 = wid * rows

    s_vs = [s_v0, s_v1]
    idx_vs = [[ix0a, ix0b], [ix1a, ix1b], [ix2a, ix2b], [ix3a, ix3b]]
    pv_vs = [[pv0a, pv0b], [pv1a, pv1b]]
    w_vs = [w_v0, w_v1]
    sem_s = [sem_s0, sem_s1]
    sem_i = [sem_i0, sem_i1]
    sem_p = [sem_p0, sem_p1]
    sem_o = [sem_o0, sem_o1]

    pltpu.sync_copy(rpi_hbm.at[pl.ds(i0, rows), :], rpi_v)
    pltpu.sync_copy(rpb_hbm, rpb_v)
    pltpu.sync_copy(shift_hbm, shift_v)
    in_plane0 = shift_v[0:16] == 0

    zeros16 = jnp.zeros((16,), jnp.float32)
    for p in range(2):
        for r in range(rows):
            for j0 in range(0, n, 16):
                w_vs[p][r, j0:j0 + 16] = zeros16

    ngroups = topk // 16
    nbh = b_ * H
    row_slice = pl.ds(i0, rows)

    def sel_idx(q, r, g):
        sl = slice(g * 16, (g + 1) * 16)
        return jnp.where(in_plane0, idx_vs[q][0][r, sl], idx_vs[q][1][r, sl])

    def fire_inputs(bh, q):
        p = q % 2
        b = bh // H
        h = bh % H
        pltpu.async_copy(s_hbm.at[b, h, row_slice, :], s_vs[p], sem_s[p])
        for sh in range(2):
            pltpu.async_copy(idx_hbm.at[sh, b, h, row_slice, :],
                             idx_vs[q][sh], sem_i[p])
            pltpu.async_copy(pv_hbm.at[sh, b, h, row_slice, :],
                             pv_vs[p][sh], sem_p[p])

    def wait_inputs(bh, q):
        p = q % 2
        b = bh // H
        h = bh % H
        pltpu.make_async_copy(s_hbm.at[b, h, row_slice, :], s_vs[p],
                              sem_s[p]).wait()
        for sh in range(2):
            pltpu.make_async_copy(idx_hbm.at[sh, b, h, row_slice, :],
                                  idx_vs[q][sh], sem_i[p]).wait()
            pltpu.make_async_copy(pv_hbm.at[sh, b, h, row_slice, :],
                                  pv_vs[p][sh], sem_p[p]).wait()

    fire_inputs(0, 0)
    fire_inputs(1, 1)

    def bh_iter(bh, q):
        p = q % 2
        b = bh // H
        h = bh % H

        @pl.when(bh >= 2)
        def _drain_and_rezero():
            bm = (bh - 2) // H
            hm = (bh - 2) % H
            pltpu.make_async_copy(w_vs[p], w_hbm.at[bm, hm, row_slice, :],
                                  sem_o[p]).wait()
            oq = (q + 2) % 4
            for r in range(rows):
                r16 = jnp.full((16,), r, jnp.int32)
                for g in range(ngroups):
                    plsc.store_scatter(w_vs[p], [r16, sel_idx(oq, r, g)],
                                       zeros16)

        wait_inputs(bh, q)
        h16 = jnp.full((16,), h, jnp.int32)
        for r in range(rows):
            r16 = jnp.full((16,), r, jnp.int32)
            ws = []
            idxs = []
            tot = zeros16
            for g in range(ngroups):
                sl = slice(g * 16, (g + 1) * 16)
                idx_g = sel_idx(q, r, g)
                s_g = plsc.load_gather(s_vs[p], [r16, idx_g])
                ri_g = plsc.load_gather(rpi_v, [r16, idx_g])
                rb_g = plsc.load_gather(rpb_v, [ri_g, h16])
                pv_g = jnp.where(in_plane0, pv_vs[p][0][r, sl],
                                 pv_vs[p][1][r, sl])
                e_g = jnp.exp(s_g + rb_g) * pv_g
                tot = tot + e_g
                ws.append(e_g)
                idxs.append(idx_g)
            den = jnp.full((16,), jnp.sum(tot) + 1e-20, jnp.float32)
            for g in range(ngroups):
                plsc.addupdate_scatter(w_vs[p], [r16, idxs[g]], ws[g] / den)
        pltpu.async_copy(w_vs[p], w_hbm.at[b, h, row_slice, :], sem_o[p])

        @pl.when(bh + 2 < nbh)
        def _prefetch():
            fire_inputs(bh + 2, (q + 2) % 4)

    def outer(t, carry):
        for qq in range(4):
            bh_iter(t * 4 + qq, qq)
        return carry

    lax.fori_loop(0, nbh // 4, outer, 0)

    for p in range(2):
        bh = nbh - 2 + p
        b = bh // H
        h = bh % H
        pltpu.make_async_copy(w_vs[p], w_hbm.at[b, h, row_slice, :],
                              sem_o[p]).wait()


def _make_sc_attn(b_, n, H, topk, nrpb):
    rows = n // 32
    mesh = plsc.VectorSubcoreMesh(core_axis_name="c", subcore_axis_name="s",
                                  num_cores=2, num_subcores=16)
    idx_buf = pltpu.VMEM((rows, topk), jnp.int32)
    pv_buf = pltpu.VMEM((rows, topk), jnp.float32)
    return pl.kernel(
        functools.partial(_sc_attn_body, b_=b_, n=n, H=H, topk=topk,
                          rows=rows),
        out_type=jax.ShapeDtypeStruct((b_, H, n, n), jnp.float32),
        mesh=mesh,
        compiler_params=pltpu.CompilerParams(use_tc_tiling_on_sc=False,
                                             needs_layout_passes=False),
        scratch_types=[
            pltpu.VMEM((rows, n), jnp.float32),
            pltpu.VMEM((rows, n), jnp.float32),
            idx_buf, idx_buf, idx_buf, idx_buf,
            idx_buf, idx_buf, idx_buf, idx_buf,
            pv_buf, pv_buf, pv_buf, pv_buf,
            pltpu.VMEM((rows, n), jnp.float32),
            pltpu.VMEM((rows, n), jnp.float32),
            pltpu.VMEM((rows, n), jnp.int32),
            pltpu.VMEM((nrpb, H), jnp.float32),
            pltpu.VMEM((16,), jnp.int32),
            pltpu.SemaphoreType.DMA,
            pltpu.SemaphoreType.DMA,
            pltpu.SemaphoreType.DMA,
            pltpu.SemaphoreType.DMA,
            pltpu.SemaphoreType.DMA,
            pltpu.SemaphoreType.DMA,
            pltpu.SemaphoreType.DMA,
            pltpu.SemaphoreType.DMA,
        ],
    )


def _out_kernel(w_ref, qkvp_ref, pw_ref, pb_ref, o_ref, *, c, H, dh):
    blk = qkvp_ref[0]
    parts = []
    for h in range(H):
        v = blk[:, 2 * c + h * dh:2 * c + (h + 1) * dh]
        parts.append(lax.dot_general(
            w_ref[0, h], v,
            (((1,), (0,)), ((), ())), preferred_element_type=jnp.float32))
    x = jnp.concatenate(parts, axis=1) + blk[:, 3 * c:4 * c]
    o = lax.dot_general(x, pw_ref[...], (((1,), (1,)), ((), ())),
                        preferred_element_type=jnp.float32)
    o_ref[0] = o + pb_ref[...]


def _make_out(b_, n, c, H, dh):
    return pl.pallas_call(
        functools.partial(_out_kernel, c=c, H=H, dh=dh),
        grid=(b_,),
        in_specs=[
            pl.BlockSpec((1, H, n, n), lambda b: (b, 0, 0, 0)),
            pl.BlockSpec((1, n, 4 * c), lambda b: (b, 0, 0)),
            pl.BlockSpec((c, c), lambda b: (0, 0)),
            pl.BlockSpec((1, c), lambda b: (0, 0)),
        ],
        out_specs=pl.BlockSpec((1, n, c), lambda b: (b, 0, 0)),
        out_shape=jax.ShapeDtypeStruct((b_, n, c), jnp.float32),
    )


def kernel(qkvp, pfa_values, pfa_indices, rpi, rpb_table, proj_w, proj_b,
           shift):
    b_, n, c4 = qkvp.shape
    c = c4 // 4
    H = rpb_table.shape[1]
    dh = c // H
    topk = pfa_indices.shape[-1]
    nrpb = rpb_table.shape[0]
    scale = dh ** (-0.5)

    shift_arr = jnp.full((16,), shift, jnp.int32)

    s = _make_scores(b_, n, c, H, dh, scale)(qkvp)
    w = _make_sc_attn(b_, n, H, topk, nrpb)(
        s, pfa_indices.astype(jnp.int32), pfa_values,
        rpi.astype(jnp.int32), rpb_table, shift_arr)
    out = _make_out(b_, n, c, H, dh)(
        w, qkvp, proj_w, proj_b.reshape(1, c))
    return out


# speedup vs baseline: 127.8579x; 1.1673x over previous
import functools

import jax
import jax.numpy as jnp
from jax import lax
from jax.experimental import pallas as pl
from jax.experimental.pallas import tpu as pltpu
from jax.experimental.pallas import tpu_sc as plsc


def _scores_kernel(qkvp_ref, s_ref, *, c, H, dh, scale):
    blk = qkvp_ref[0]
    for h in range(H):
        q = blk[:, h * dh:(h + 1) * dh]
        k = blk[:, c + h * dh:c + (h + 1) * dh]
        for par in range(2):
            kp = k[par * 128:(par + 1) * 128, :]
            s = lax.dot_general(q, kp, (((1,), (1,)), ((), ())),
                                preferred_element_type=jnp.float32)
            s_ref[h, par] = s * scale


def _make_scores(b_, n, c, H, dh, scale):
    return pl.pallas_call(
        functools.partial(_scores_kernel, c=c, H=H, dh=dh, scale=scale),
        grid=(b_,),
        in_specs=[
            pl.BlockSpec((1, n, 4 * c), lambda b: (b, 0, 0)),
        ],
        out_specs=pl.BlockSpec((H, 2, n, 128), lambda b: (b, 0, 0, 0)),
        out_shape=jax.ShapeDtypeStruct((b_ * H, 2, n, 128), jnp.float32),
    )


def _sc_attn_body(s_hbm, idx_hbm, pv_hbm, rpi_hbm, rpb_hbm, shift_hbm, w_hbm,
                  s_v0, s_v1,
                  ix0a, ix0b, ix1a, ix1b, ix2a, ix2b, ix3a, ix3b,
                  pv0a, pv0b, pv1a, pv1b,
                  w_v0, w_v1, rpi_v, rpb_v, shift_v,
                  sem_s0, sem_s1, sem_i0, sem_i1, sem_p0, sem_p1,
                  sem_o0, sem_o1,
                  *, b_, n, H, topk, rows):
    cid = lax.axis_index("c")
    sid = lax.axis_index("s")
    wid = sid * 2 + cid
    i0 = wid * rows

    s_vs = [s_v0, s_v1]
    idx_vs = [[ix0a, ix0b], [ix1a, ix1b], [ix2a, ix2b], [ix3a, ix3b]]
    pv_vs = [[pv0a, pv0b], [pv1a, pv1b]]
    w_vs = [w_v0, w_v1]
    sem_s = [sem_s0, sem_s1]
    sem_i = [sem_i0, sem_i1]
    sem_p = [sem_p0, sem_p1]
    sem_o = [sem_o0, sem_o1]

    pltpu.sync_copy(rpi_hbm.at[pl.ds(i0 * 2, rows * 2), :], rpi_v)
    pltpu.sync_copy(rpb_hbm, rpb_v)
    pltpu.sync_copy(shift_hbm, shift_v)
    in_plane0 = shift_v[0:16] == 0

    zeros16 = jnp.zeros((16,), jnp.float32)
    for p in range(2):
        for par in range(2):
            for r in range(rows):
                for j0 in range(0, 128, 16):
                    w_vs[p][par, r, j0:j0 + 16] = zeros16

    ngroups = topk // 16
    nbh = b_ * H
    ir0 = (i0 * topk) // 128
    irows = (rows * topk) // 128

    def sel_vec(bufs, r, g):
        flat = r * topk + g * 16
        fr, fc = flat // 128, flat % 128
        return jnp.where(in_plane0, bufs[0][fr, fc:fc + 16],
                         bufs[1][fr, fc:fc + 16])

    def fire_inputs(bh, q):
        p = q % 2
        pltpu.async_copy(s_hbm.at[bh, :, pl.ds(i0, rows), :], s_vs[p],
                         sem_s[p])
        for sh in range(2):
            pltpu.async_copy(idx_hbm.at[sh, bh, pl.ds(ir0, irows), :],
                             idx_vs[q][sh], sem_i[p])
            pltpu.async_copy(pv_hbm.at[sh, bh, pl.ds(ir0, irows), :],
                             pv_vs[p][sh], sem_p[p])

    def wait_inputs(bh, q):
        p = q % 2
        pltpu.make_async_copy(s_hbm.at[bh, :, pl.ds(i0, rows), :], s_vs[p],
                              sem_s[p]).wait()
        for sh in range(2):
            pltpu.make_async_copy(idx_hbm.at[sh, bh, pl.ds(ir0, irows), :],
                                  idx_vs[q][sh], sem_i[p]).wait()
            pltpu.make_async_copy(pv_hbm.at[sh, bh, pl.ds(ir0, irows), :],
                                  pv_vs[p][sh], sem_p[p]).wait()

    fire_inputs(0, 0)
    fire_inputs(1, 1)

    c127 = jnp.full((16,), 127, jnp.int32)

    def bh_iter(bh, q):
        p = q % 2
        h = bh % H

        @pl.when(bh >= 2)
        def _drain_and_rezero():
            pltpu.make_async_copy(w_vs[p],
                                  w_hbm.at[bh - 2, :, pl.ds(i0, rows), :],
                                  sem_o[p]).wait()
            oq = (q + 2) % 4
            for r in range(rows):
                r16 = jnp.full((16,), r, jnp.int32)
                for g in range(ngroups):
                    oidx = sel_vec(idx_vs[oq], r, g)
                    plsc.store_scatter(
                        w_vs[p],
                        [lax.shift_right_logical(oidx, 7), r16,
                         lax.bitwise_and(oidx, c127)],
                        zeros16)

        wait_inputs(bh, q)
        h16 = jnp.full((16,), h, jnp.int32)
        for r in range(rows):
            r16 = jnp.full((16,), r, jnp.int32)
            r16x2 = jnp.full((16,), 2 * r, jnp.int32)
            ws = []
            jhis = []
            jlos = []
            tot = zeros16
            for g in range(ngroups):
                idx_g = sel_vec(idx_vs[q], r, g)
                jhi = lax.shift_right_logical(idx_g, 7)
                jlo = lax.bitwise_and(idx_g, c127)
                s_g = plsc.load_gather(s_vs[p], [jhi, r16, jlo])
                ri_g = plsc.load_gather(rpi_v, [r16x2 + jhi, jlo])
                f_g = ri_g * H + h16
                rb_g = plsc.load_gather(
                    rpb_v, [lax.shift_right_logical(f_g, 7),
                            lax.bitwise_and(f_g, c127)])
                pv_g = sel_vec(pv_vs[p], r, g)
                e_g = jnp.exp(s_g + rb_g) * pv_g
                tot = tot + e_g
                ws.append(e_g)
                jhis.append(jhi)
                jlos.append(jlo)
            den = jnp.full((16,), jnp.sum(tot) + 1e-20, jnp.float32)
            for g in range(ngroups):
                plsc.addupdate_scatter(w_vs[p], [jhis[g], r16, jlos[g]],
                                       ws[g] / den)
        pltpu.async_copy(w_vs[p], w_hbm.at[bh, :, pl.ds(i0, rows), :],
                         sem_o[p])

        @pl.when(bh + 2 < nbh)
        def _prefetch():
            fire_inputs(bh + 2, (q + 2) % 4)

    def outer(t, carry):
        for qq in range(4):
            bh_iter(t * 4 + qq, qq)
        return carry

    lax.fori_loop(0, nbh // 4, outer, 0)

    for p in range(2):
        bh = nbh - 2 + p
        pltpu.make_async_copy(w_vs[p], w_hbm.at[bh, :, pl.ds(i0, rows), :],
                              sem_o[p]).wait()


def _make_sc_attn(b_, n, H, topk, nrpbf):
    rows = n // 32
    irows = (rows * topk) // 128
    mesh = plsc.VectorSubcoreMesh(core_axis_name="c", subcore_axis_name="s",
                                  num_cores=2, num_subcores=16)
    idx_buf = pltpu.VMEM((irows, 128), jnp.int32)
    pv_buf = pltpu.VMEM((irows, 128), jnp.float32)
    return pl.kernel(
        functools.partial(_sc_attn_body, b_=b_, n=n, H=H, topk=topk,
                          rows=rows),
        out_type=jax.ShapeDtypeStruct((b_ * H, 2, n, 128), jnp.float32),
        mesh=mesh,
        compiler_params=pltpu.CompilerParams(use_tc_tiling_on_sc=False,
                                             needs_layout_passes=False),
        scratch_types=[
            pltpu.VMEM((2, rows, 128), jnp.float32),
            pltpu.VMEM((2, rows, 128), jnp.float32),
            idx_buf, idx_buf, idx_buf, idx_buf,
            idx_buf, idx_buf, idx_buf, idx_buf,
            pv_buf, pv_buf, pv_buf, pv_buf,
            pltpu.VMEM((2, rows, 128), jnp.float32),
            pltpu.VMEM((2, rows, 128), jnp.float32),
            pltpu.VMEM((2 * rows, 128), jnp.int32),
            pltpu.VMEM((nrpbf, 128), jnp.float32),
            pltpu.VMEM((16,), jnp.int32),
            pltpu.SemaphoreType.DMA,
            pltpu.SemaphoreType.DMA,
            pltpu.SemaphoreType.DMA,
            pltpu.SemaphoreType.DMA,
            pltpu.SemaphoreType.DMA,
            pltpu.SemaphoreType.DMA,
            pltpu.SemaphoreType.DMA,
            pltpu.SemaphoreType.DMA,
        ],
    )


def _out_kernel(w_ref, qkvp_ref, pw_ref, pb_ref, o_ref, *, c, H, dh):
    blk = qkvp_ref[0]
    parts = []
    for h in range(H):
        v = blk[:, 2 * c + h * dh:2 * c + (h + 1) * dh]
        xh = lax.dot_general(w_ref[h, 0], v[0:128, :],
                             (((1,), (0,)), ((), ())),
                             preferred_element_type=jnp.float32)
        xh = xh + lax.dot_general(w_ref[h, 1], v[128:256, :],
                                  (((1,), (0,)), ((), ())),
                                  preferred_element_type=jnp.float32)
        parts.append(xh)
    x = jnp.concatenate(parts, axis=1) + blk[:, 3 * c:4 * c]
    o = lax.dot_general(x, pw_ref[...], (((1,), (1,)), ((), ())),
                        preferred_element_type=jnp.float32)
    o_ref[0] = o + pb_ref[...]


def _make_out(b_, n, c, H, dh):
    return pl.pallas_call(
        functools.partial(_out_kernel, c=c, H=H, dh=dh),
        grid=(b_,),
        in_specs=[
            pl.BlockSpec((H, 2, n, 128), lambda b: (b, 0, 0, 0)),
            pl.BlockSpec((1, n, 4 * c), lambda b: (b, 0, 0)),
            pl.BlockSpec((c, c), lambda b: (0, 0)),
            pl.BlockSpec((1, c), lambda b: (0, 0)),
        ],
        out_specs=pl.BlockSpec((1, n, c), lambda b: (b, 0, 0)),
        out_shape=jax.ShapeDtypeStruct((b_, n, c), jnp.float32),
    )


def kernel(qkvp, pfa_values, pfa_indices, rpi, rpb_table, proj_w, proj_b,
           shift):
    b_, n, c4 = qkvp.shape
    c = c4 // 4
    H = rpb_table.shape[1]
    dh = c // H
    topk = pfa_indices.shape[-1]
    nrpb = rpb_table.shape[0]
    nrpbf = -(-(nrpb * H) // 128)
    scale = dh ** (-0.5)

    shift_arr = jnp.full((16,), shift, jnp.int32)
    idxr = pfa_indices.astype(jnp.int32).reshape(
        2, b_ * H, (n * topk) // 128, 128)
    pvr = pfa_values.reshape(2, b_ * H, (n * topk) // 128, 128)
    rpir = rpi.astype(jnp.int32).reshape((n * n) // 128, 128)
    rpbf = jnp.pad(rpb_table.reshape(-1),
                   (0, nrpbf * 128 - nrpb * H)).reshape(nrpbf, 128)

    s = _make_scores(b_, n, c, H, dh, scale)(qkvp)
    w = _make_sc_attn(b_, n, H, topk, nrpbf)(
        s, idxr, pvr, rpir, rpbf, shift_arr)
    out = _make_out(b_, n, c, H, dh)(
        w, qkvp, proj_w, proj_b.reshape(1, c))
    return out


# speedup vs baseline: 138.3013x; 1.0817x over previous
import functools

import jax
import jax.numpy as jnp
from jax import lax
from jax.experimental import pallas as pl
from jax.experimental.pallas import tpu as pltpu
from jax.experimental.pallas import tpu_sc as plsc


def _scores_kernel(shift_ref, qkvp_ref, idx_ref, pv_ref,
                   s_ref, idxo_ref, pvo_ref, *, c, H, dh, scale, n, topk):
    blk = qkvp_ref[0]
    for h in range(H):
        q = blk[:, h * dh:(h + 1) * dh]
        k = blk[:, c + h * dh:c + (h + 1) * dh]
        for par in range(2):
            kp = k[par * 128:(par + 1) * 128, :]
            s = lax.dot_general(q, kp, (((1,), (1,)), ((), ())),
                                preferred_element_type=jnp.float32)
            s_ref[h, par] = s * scale
    for h in range(H):
        for m in range(128 // topk):
            idxo_ref[h, :, m * topk:(m + 1) * topk] = (
                idx_ref[0, 0, h, m::(128 // topk), :])
            pvo_ref[h, :, m * topk:(m + 1) * topk] = (
                pv_ref[0, 0, h, m::(128 // topk), :])


def _make_scores(b_, n, c, H, dh, topk, scale):
    nr = (n * topk) // 128
    return pl.pallas_call(
        functools.partial(_scores_kernel, c=c, H=H, dh=dh, scale=scale,
                          n=n, topk=topk),
        grid_spec=pltpu.PrefetchScalarGridSpec(
            num_scalar_prefetch=1,
            grid=(b_,),
            in_specs=[
                pl.BlockSpec((1, n, 4 * c), lambda b, sref: (b, 0, 0)),
                pl.BlockSpec((1, 1, H, n, topk),
                             lambda b, sref: (sref[0], b, 0, 0, 0)),
                pl.BlockSpec((1, 1, H, n, topk),
                             lambda b, sref: (sref[0], b, 0, 0, 0)),
            ],
            out_specs=[
                pl.BlockSpec((H, 2, n, 128), lambda b, sref: (b, 0, 0, 0)),
                pl.BlockSpec((H, nr, 128), lambda b, sref: (b, 0, 0)),
                pl.BlockSpec((H, nr, 128), lambda b, sref: (b, 0, 0)),
            ],
        ),
        out_shape=[
            jax.ShapeDtypeStruct((b_ * H, 2, n, 128), jnp.float32),
            jax.ShapeDtypeStruct((b_ * H, nr, 128), jnp.int32),
            jax.ShapeDtypeStruct((b_ * H, nr, 128), jnp.float32),
        ],
    )


def _sc_attn_body(s_hbm, idx_hbm, pv_hbm, rpi_hbm, rpb_hbm, w_hbm,
                  s_v0, s_v1, ix0, ix1, ix2, ix3, pv_v0, pv_v1,
                  w_v0, w_v1, rpi_v, rpb_v, r_v,
                  sem_s0, sem_s1, sem_i0, sem_i1, sem_p0, sem_p1,
                  sem_o0, sem_o1,
                  *, b_, n, H, topk, rows):
    cid = lax.axis_index("c")
    sid = lax.axis_index("s")
    wid = sid * 2 + cid
    i0 = wid * rows

    s_vs = [s_v0, s_v1]
    idx_vs = [ix0, ix1, ix2, ix3]
    pv_vs = [pv_v0, pv_v1]
    w_vs = [w_v0, w_v1]
    sem_s = [sem_s0, sem_s1]
    sem_i = [sem_i0, sem_i1]
    sem_p = [sem_p0, sem_p1]
    sem_o = [sem_o0, sem_o1]

    pltpu.sync_copy(rpi_hbm.at[pl.ds(i0 * 2, rows * 2), :], rpi_v)
    pltpu.sync_copy(rpb_hbm, rpb_v)

    for h in range(H):
        h16 = jnp.full((16,), h, jnp.int32)
        for rr in range(2 * rows):
            for g in range(128 // 16):
                ri = rpi_v[rr, g * 16:(g + 1) * 16]
                f = ri * H + h16
                rb = plsc.load_gather(
                    rpb_v, [lax.shift_right_logical(f, 7),
                            lax.bitwise_and(f, jnp.full((16,), 127,
                                                        jnp.int32))])
                r_v[h * 2 * rows + rr, g * 16:(g + 1) * 16] = rb

    zeros16 = jnp.zeros((16,), jnp.float32)
    for p in range(2):
        for par in range(2):
            for r in range(rows):
                for j0 in range(0, 128, 16):
                    w_vs[p][par, r, j0:j0 + 16] = zeros16

    ngroups = topk // 16
    nbh = b_ * H
    ir0 = (i0 * topk) // 128
    irows = (rows * topk) // 128

    def slc(buf, r, g):
        flat = r * topk + g * 16
        return buf[flat // 128, flat % 128:flat % 128 + 16]

    def fire_inputs(bh, q):
        p = q % 2
        pltpu.async_copy(s_hbm.at[bh, :, pl.ds(i0, rows), :], s_vs[p],
                         sem_s[p])
        pltpu.async_copy(idx_hbm.at[bh, pl.ds(ir0, irows), :],
                         idx_vs[q], sem_i[p])
        pltpu.async_copy(pv_hbm.at[bh, pl.ds(ir0, irows), :],
                         pv_vs[p], sem_p[p])

    def wait_inputs(bh, q):
        p = q % 2
        pltpu.make_async_copy(s_hbm.at[bh, :, pl.ds(i0, rows), :], s_vs[p],
                              sem_s[p]).wait()
        pltpu.make_async_copy(idx_hbm.at[bh, pl.ds(ir0, irows), :],
                              idx_vs[q], sem_i[p]).wait()
        pltpu.make_async_copy(pv_hbm.at[bh, pl.ds(ir0, irows), :],
                              pv_vs[p], sem_p[p]).wait()

    fire_inputs(0, 0)
    fire_inputs(1, 1)

    c127 = jnp.full((16,), 127, jnp.int32)

    def bh_iter(bh, q):
        p = q % 2
        h = bh % H

        @pl.when(bh >= 2)
        def _drain_and_rezero():
            pltpu.make_async_copy(w_vs[p],
                                  w_hbm.at[bh - 2, :, pl.ds(i0, rows), :],
                                  sem_o[p]).wait()
            oidx = idx_vs[(q + 2) % 4]
            for r in range(rows):
                r16 = jnp.full((16,), r, jnp.int32)
                for g in range(ngroups):
                    og = slc(oidx, r, g)
                    plsc.store_scatter(
                        w_vs[p],
                        [lax.shift_right_logical(og, 7), r16,
                         lax.bitwise_and(og, c127)],
                        zeros16)

        wait_inputs(bh, q)
        hbase = jnp.full((16,), h * 2 * rows, jnp.int32)
        for r in range(rows):
            r16 = jnp.full((16,), r, jnp.int32)
            rbase = hbase + (2 * r)
            for g in range(ngroups):
                idx_g = slc(idx_vs[q], r, g)
                jhi = lax.shift_right_logical(idx_g, 7)
                jlo = lax.bitwise_and(idx_g, c127)
                s_g = plsc.load_gather(s_vs[p], [jhi, r16, jlo])
                rb_g = plsc.load_gather(r_v, [rbase + jhi, jlo])
                e_g = jnp.exp(s_g + rb_g) * slc(pv_vs[p], r, g)
                plsc.addupdate_scatter(w_vs[p], [jhi, r16, jlo], e_g)
        pltpu.async_copy(w_vs[p], w_hbm.at[bh, :, pl.ds(i0, rows), :],
                         sem_o[p])

        @pl.when(bh + 2 < nbh)
        def _prefetch():
            fire_inputs(bh + 2, (q + 2) % 4)

    def outer(t, carry):
        for qq in range(4):
            bh_iter(t * 4 + qq, qq)
        return carry

    lax.fori_loop(0, nbh // 4, outer, 0)

    for p in range(2):
        bh = nbh - 2 + p
        pltpu.make_async_copy(w_vs[p], w_hbm.at[bh, :, pl.ds(i0, rows), :],
                              sem_o[p]).wait()


def _make_sc_attn(b_, n, H, topk, nrpbf):
    rows = n // 32
    irows = (rows * topk) // 128
    mesh = plsc.VectorSubcoreMesh(core_axis_name="c", subcore_axis_name="s",
                                  num_cores=2, num_subcores=16)
    idx_buf = pltpu.VMEM((irows, 128), jnp.int32)
    pv_buf = pltpu.VMEM((irows, 128), jnp.float32)
    return pl.kernel(
        functools.partial(_sc_attn_body, b_=b_, n=n, H=H, topk=topk,
                          rows=rows),
        out_type=jax.ShapeDtypeStruct((b_ * H, 2, n, 128), jnp.float32),
        mesh=mesh,
        compiler_params=pltpu.CompilerParams(use_tc_tiling_on_sc=False,
                                             needs_layout_passes=False),
        scratch_types=[
            pltpu.VMEM((2, rows, 128), jnp.float32),
            pltpu.VMEM((2, rows, 128), jnp.float32),
            idx_buf, idx_buf, idx_buf, idx_buf,
            pv_buf, pv_buf,
            pltpu.VMEM((2, rows, 128), jnp.float32),
            pltpu.VMEM((2, rows, 128), jnp.float32),
            pltpu.VMEM((2 * rows, 128), jnp.int32),
            pltpu.VMEM((nrpbf, 128), jnp.float32),
            pltpu.VMEM((H * 2 * rows, 128), jnp.float32),
            pltpu.SemaphoreType.DMA,
            pltpu.SemaphoreType.DMA,
            pltpu.SemaphoreType.DMA,
            pltpu.SemaphoreType.DMA,
            pltpu.SemaphoreType.DMA,
            pltpu.SemaphoreType.DMA,
            pltpu.SemaphoreType.DMA,
            pltpu.SemaphoreType.DMA,
        ],
    )


def _out_kernel(w_ref, qkvp_ref, pw_ref, pb_ref, o_ref, *, c, H, dh):
    blk = qkvp_ref[0]
    parts = []
    for h in range(H):
        v = blk[:, 2 * c + h * dh:2 * c + (h + 1) * dh]
        w0 = w_ref[h, 0]
        w1 = w_ref[h, 1]
        den = (jnp.sum(w0, axis=1, keepdims=True) +
               jnp.sum(w1, axis=1, keepdims=True) + 1e-20)
        xh = lax.dot_general(w0, v[0:128, :], (((1,), (0,)), ((), ())),
                             preferred_element_type=jnp.float32)
        xh = xh + lax.dot_general(w1, v[128:256, :],
                                  (((1,), (0,)), ((), ())),
                                  preferred_element_type=jnp.float32)
        parts.append(xh / den)
    x = jnp.concatenate(parts, axis=1) + blk[:, 3 * c:4 * c]
    o = lax.dot_general(x, pw_ref[...], (((1,), (1,)), ((), ())),
                        preferred_element_type=jnp.float32)
    o_ref[0] = o + pb_ref[...]


def _make_out(b_, n, c, H, dh):
    return pl.pallas_call(
        functools.partial(_out_kernel, c=c, H=H, dh=dh),
        grid=(b_,),
        in_specs=[
            pl.BlockSpec((H, 2, n, 128), lambda b: (b, 0, 0, 0)),
            pl.BlockSpec((1, n, 4 * c), lambda b: (b, 0, 0)),
            pl.BlockSpec((c, c), lambda b: (0, 0)),
            pl.BlockSpec((1, c), lambda b: (0, 0)),
        ],
        out_specs=pl.BlockSpec((1, n, c), lambda b: (b, 0, 0)),
        out_shape=jax.ShapeDtypeStruct((b_, n, c), jnp.float32),
    )


def kernel(qkvp, pfa_values, pfa_indices, rpi, rpb_table, proj_w, proj_b,
           shift):
    b_, n, c4 = qkvp.shape
    c = c4 // 4
    H = rpb_table.shape[1]
    dh = c // H
    topk = pfa_indices.shape[-1]
    nrpb = rpb_table.shape[0]
    nrpbf = -(-(nrpb * H) // 128)
    scale = dh ** (-0.5)

    shift_arr = jnp.asarray(shift, jnp.int32).reshape(1)
    rpir = rpi.astype(jnp.int32).reshape((n * n) // 128, 128)
    rpbf = jnp.pad(rpb_table.reshape(-1),
                   (0, nrpbf * 128 - nrpb * H)).reshape(nrpbf, 128)

    s, idxr, pvr = _make_scores(b_, n, c, H, dh, topk, scale)(
        shift_arr, qkvp, pfa_indices.astype(jnp.int32), pfa_values)
    w = _make_sc_attn(b_, n, H, topk, nrpbf)(s, idxr, pvr, rpir, rpbf)
    out = _make_out(b_, n, c, H, dh)(w, qkvp, proj_w, proj_b.reshape(1, c))
    return out


# speedup vs baseline: 146.1355x; 1.0566x over previous
import functools

import jax
import jax.numpy as jnp
from jax import lax
from jax.experimental import pallas as pl
from jax.experimental.pallas import tpu as pltpu
from jax.experimental.pallas import tpu_sc as plsc


def _scores_kernel(shift_ref, qkvp_ref, idx_ref, pv_ref,
                   s_ref, idxo_ref, pvo_ref, *, c, H, dh, scale, n, topk):
    blk = qkvp_ref[0]
    for h in range(H):
        q = blk[:, h * dh:(h + 1) * dh]
        k = blk[:, c + h * dh:c + (h + 1) * dh]
        for par in range(2):
            kp = k[par * 128:(par + 1) * 128, :]
            s = lax.dot_general(q, kp, (((1,), (1,)), ((), ())),
                                preferred_element_type=jnp.float32)
            s_ref[h, par] = s * scale
    for h in range(H):
        for m in range(128 // topk):
            idxo_ref[h, :, m * topk:(m + 1) * topk] = (
                idx_ref[0, 0, h, m::(128 // topk), :])
            pvo_ref[h, :, m * topk:(m + 1) * topk] = (
                pv_ref[0, 0, h, m::(128 // topk), :])


def _make_scores(bc, boff, n, c, H, dh, topk, scale):
    nr = (n * topk) // 128
    return pl.pallas_call(
        functools.partial(_scores_kernel, c=c, H=H, dh=dh, scale=scale,
                          n=n, topk=topk),
        grid_spec=pltpu.PrefetchScalarGridSpec(
            num_scalar_prefetch=1,
            grid=(bc,),
            in_specs=[
                pl.BlockSpec((1, n, 2 * c), lambda b, sref: (b + boff, 0, 0)),
                pl.BlockSpec((1, 1, H, n, topk),
                             lambda b, sref: (sref[0], b + boff, 0, 0, 0)),
                pl.BlockSpec((1, 1, H, n, topk),
                             lambda b, sref: (sref[0], b + boff, 0, 0, 0)),
            ],
            out_specs=[
                pl.BlockSpec((H, 2, n, 128), lambda b, sref: (b, 0, 0, 0)),
                pl.BlockSpec((H, nr, 128), lambda b, sref: (b, 0, 0)),
                pl.BlockSpec((H, nr, 128), lambda b, sref: (b, 0, 0)),
            ],
        ),
        out_shape=[
            jax.ShapeDtypeStruct((bc * H, 2, n, 128), jnp.float32),
            jax.ShapeDtypeStruct((bc * H, nr, 128), jnp.int32),
            jax.ShapeDtypeStruct((bc * H, nr, 128), jnp.float32),
        ],
    )


def _sc_attn_body(s_hbm, idx_hbm, pv_hbm, rpi_hbm, rpb_hbm, w_hbm,
                  s_v0, s_v1, ix0, ix1, ix2, ix3, pv_v0, pv_v1,
                  w_v0, w_v1, rpi_v, rpb_v, r_v,
                  sem_s0, sem_s1, sem_i0, sem_i1, sem_p0, sem_p1,
                  sem_o0, sem_o1,
                  *, b_, n, H, topk, rows):
    cid = lax.axis_index("c")
    sid = lax.axis_index("s")
    wid = sid * 2 + cid
    i0 = wid * rows

    s_vs = [s_v0, s_v1]
    idx_vs = [ix0, ix1, ix2, ix3]
    pv_vs = [pv_v0, pv_v1]
    w_vs = [w_v0, w_v1]
    sem_s = [sem_s0, sem_s1]
    sem_i = [sem_i0, sem_i1]
    sem_p = [sem_p0, sem_p1]
    sem_o = [sem_o0, sem_o1]

    pltpu.sync_copy(rpi_hbm.at[pl.ds(i0 * 2, rows * 2), :], rpi_v)
    pltpu.sync_copy(rpb_hbm, rpb_v)

    for h in range(H):
        h16 = jnp.full((16,), h, jnp.int32)
        for rr in range(2 * rows):
            for g in range(128 // 16):
                ri = rpi_v[rr, g * 16:(g + 1) * 16]
                f = ri * H + h16
                rb = plsc.load_gather(
                    rpb_v, [lax.shift_right_logical(f, 7),
                            lax.bitwise_and(f, jnp.full((16,), 127,
                                                        jnp.int32))])
                r_v[h * 2 * rows + rr, g * 16:(g + 1) * 16] = rb

    zeros16 = jnp.zeros((16,), jnp.float32)
    for p in range(2):
        for par in range(2):
            for r in range(rows):
                for j0 in range(0, 128, 16):
                    w_vs[p][par, r, j0:j0 + 16] = zeros16

    ngroups = topk // 16
    nbh = b_ * H
    ir0 = (i0 * topk) // 128
    irows = (rows * topk) // 128

    def slc(buf, r, g):
        flat = r * topk + g * 16
        return buf[flat // 128, flat % 128:flat % 128 + 16]

    def fire_inputs(bh, q):
        p = q % 2
        pltpu.async_copy(s_hbm.at[bh, :, pl.ds(i0, rows), :], s_vs[p],
                         sem_s[p])
        pltpu.async_copy(idx_hbm.at[bh, pl.ds(ir0, irows), :],
                         idx_vs[q], sem_i[p])
        pltpu.async_copy(pv_hbm.at[bh, pl.ds(ir0, irows), :],
                         pv_vs[p], sem_p[p])

    def wait_inputs(bh, q):
        p = q % 2
        pltpu.make_async_copy(s_hbm.at[bh, :, pl.ds(i0, rows), :], s_vs[p],
                              sem_s[p]).wait()
        pltpu.make_async_copy(idx_hbm.at[bh, pl.ds(ir0, irows), :],
                              idx_vs[q], sem_i[p]).wait()
        pltpu.make_async_copy(pv_hbm.at[bh, pl.ds(ir0, irows), :],
                              pv_vs[p], sem_p[p]).wait()

    fire_inputs(0, 0)
    fire_inputs(1, 1)

    c127 = jnp.full((16,), 127, jnp.int32)

    def bh_iter(bh, q):
        p = q % 2
        h = bh % H

        @pl.when(bh >= 2)
        def _drain_and_rezero():
            pltpu.make_async_copy(w_vs[p],
                                  w_hbm.at[bh - 2, :, pl.ds(i0, rows), :],
                                  sem_o[p]).wait()
            oidx = idx_vs[(q + 2) % 4]
            for r in range(rows):
                r16 = jnp.full((16,), r, jnp.int32)
                for g in range(ngroups):
                    og = slc(oidx, r, g)
                    plsc.store_scatter(
                        w_vs[p],
                        [lax.shift_right_logical(og, 7), r16,
                         lax.bitwise_and(og, c127)],
                        zeros16)

        wait_inputs(bh, q)
        hbase = jnp.full((16,), h * 2 * rows, jnp.int32)
        for r in range(rows):
            r16 = jnp.full((16,), r, jnp.int32)
            rbase = hbase + (2 * r)
            for g in range(ngroups):
                idx_g = slc(idx_vs[q], r, g)
                jhi = lax.shift_right_logical(idx_g, 7)
                jlo = lax.bitwise_and(idx_g, c127)
                s_g = plsc.load_gather(s_vs[p], [jhi, r16, jlo])
                rb_g = plsc.load_gather(r_v, [rbase + jhi, jlo])
                e_g = jnp.exp(s_g + rb_g) * slc(pv_vs[p], r, g)
                plsc.addupdate_scatter(w_vs[p], [jhi, r16, jlo], e_g)
        pltpu.async_copy(w_vs[p], w_hbm.at[bh, :, pl.ds(i0, rows), :],
                         sem_o[p])

        @pl.when(bh + 2 < nbh)
        def _prefetch():
            fire_inputs(bh + 2, (q + 2) % 4)

    def outer(t, carry):
        for qq in range(4):
            bh_iter(t * 4 + qq, qq)
        return carry

    lax.fori_loop(0, nbh // 4, outer, 0)

    for p in range(2):
        bh = nbh - 2 + p
        pltpu.make_async_copy(w_vs[p], w_hbm.at[bh, :, pl.ds(i0, rows), :],
                              sem_o[p]).wait()


def _make_sc_attn(b_, n, H, topk, nrpbf):
    rows = n // 32
    irows = (rows * topk) // 128
    mesh = plsc.VectorSubcoreMesh(core_axis_name="c", subcore_axis_name="s",
                                  num_cores=2, num_subcores=16)
    idx_buf = pltpu.VMEM((irows, 128), jnp.int32)
    pv_buf = pltpu.VMEM((irows, 128), jnp.float32)
    return pl.kernel(
        functools.partial(_sc_attn_body, b_=b_, n=n, H=H, topk=topk,
                          rows=rows),
        out_type=jax.ShapeDtypeStruct((b_ * H, 2, n, 128), jnp.float32),
        mesh=mesh,
        compiler_params=pltpu.CompilerParams(use_tc_tiling_on_sc=False,
                                             needs_layout_passes=False),
        scratch_types=[
            pltpu.VMEM((2, rows, 128), jnp.float32),
            pltpu.VMEM((2, rows, 128), jnp.float32),
            idx_buf, idx_buf, idx_buf, idx_buf,
            pv_buf, pv_buf,
            pltpu.VMEM((2, rows, 128), jnp.float32),
            pltpu.VMEM((2, rows, 128), jnp.float32),
            pltpu.VMEM((2 * rows, 128), jnp.int32),
            pltpu.VMEM((nrpbf, 128), jnp.float32),
            pltpu.VMEM((H * 2 * rows, 128), jnp.float32),
            pltpu.SemaphoreType.DMA,
            pltpu.SemaphoreType.DMA,
            pltpu.SemaphoreType.DMA,
            pltpu.SemaphoreType.DMA,
            pltpu.SemaphoreType.DMA,
            pltpu.SemaphoreType.DMA,
            pltpu.SemaphoreType.DMA,
            pltpu.SemaphoreType.DMA,
        ],
    )


def _out_kernel(w_ref, qkvp_ref, pw_ref, pb_ref, o_ref, *, c, H, dh):
    blk = qkvp_ref[0]
    parts = []
    for h in range(H):
        v = blk[:, h * dh:(h + 1) * dh]
        w0 = w_ref[h, 0]
        w1 = w_ref[h, 1]
        den = (jnp.sum(w0, axis=1, keepdims=True) +
               jnp.sum(w1, axis=1, keepdims=True) + 1e-20)
        xh = lax.dot_general(w0, v[0:128, :], (((1,), (0,)), ((), ())),
                             preferred_element_type=jnp.float32)
        xh = xh + lax.dot_general(w1, v[128:256, :],
                                  (((1,), (0,)), ((), ())),
                                  preferred_element_type=jnp.float32)
        parts.append(xh / den)
    x = jnp.concatenate(parts, axis=1) + blk[:, c:2 * c]
    o = lax.dot_general(x, pw_ref[...], (((1,), (1,)), ((), ())),
                        preferred_element_type=jnp.float32)
    o_ref[0] = o + pb_ref[...]


def _make_out(bc, boff, n, c, H, dh):
    return pl.pallas_call(
        functools.partial(_out_kernel, c=c, H=H, dh=dh),
        grid=(bc,),
        in_specs=[
            pl.BlockSpec((H, 2, n, 128), lambda b: (b, 0, 0, 0)),
            pl.BlockSpec((1, n, 2 * c), lambda b: (b + boff, 0, 1)),
            pl.BlockSpec((c, c), lambda b: (0, 0)),
            pl.BlockSpec((1, c), lambda b: (0, 0)),
        ],
        out_specs=pl.BlockSpec((1, n, c), lambda b: (b, 0, 0)),
        out_shape=jax.ShapeDtypeStruct((bc, n, c), jnp.float32),
    )


def kernel(qkvp, pfa_values, pfa_indices, rpi, rpb_table, proj_w, proj_b,
           shift):
    b_, n, c4 = qkvp.shape
    c = c4 // 4
    H = rpb_table.shape[1]
    dh = c // H
    topk = pfa_indices.shape[-1]
    nrpb = rpb_table.shape[0]
    nrpbf = -(-(nrpb * H) // 128)
    scale = dh ** (-0.5)

    shift_arr = jnp.asarray(shift, jnp.int32).reshape(1)
    rpir = rpi.astype(jnp.int32).reshape((n * n) // 128, 128)
    rpbf = jnp.pad(rpb_table.reshape(-1),
                   (0, nrpbf * 128 - nrpb * H)).reshape(nrpbf, 128)
    idx32 = pfa_indices.astype(jnp.int32)
    pb2 = proj_b.reshape(1, c)

    nchunks = 2
    bc = b_ // nchunks
    outs = []
    for ci in range(nchunks):
        boff = ci * bc
        s, idxr, pvr = _make_scores(bc, boff, n, c, H, dh, topk, scale)(
            shift_arr, qkvp, idx32, pfa_values)
        w = _make_sc_attn(bc, n, H, topk, nrpbf)(s, idxr, pvr, rpir, rpbf)
        outs.append(_make_out(bc, boff, n, c, H, dh)(w, qkvp, proj_w, pb2))
    return jnp.concatenate(outs, axis=0)
